# baseline (speedup 1.0000x reference)
import jax
import jax.numpy as jnp
from jax.experimental import pallas as pl
from jax.experimental.pallas import tpu as pltpu

_C = 80
_K = 100
_OUT = 300
_IOU_T = 0.7
_N = 20000
_ROWS = 160
_LANES = 128
_POPS = 160
_NEG = float("-inf")


def _nms_body(props_ref, deltas_ref, scores_ref, sel_s_ref, sel_b_ref,
              boxes_ref, valid_ref):
    c = pl.program_id(0)

    @pl.when(c == 0)
    def _decode():
        py1 = props_ref[0]
        px1 = props_ref[1]
        py2 = props_ref[2]
        px2 = props_ref[3]
        h = py2 - py1
        w = px2 - px1
        cy = py1 + 0.5 * h
        cx = px1 + 0.5 * w
        tx = deltas_ref[0]
        ty = deltas_ref[1]
        tw = deltas_ref[2]
        th = deltas_ref[3]
        ncx = tx * w + cx
        ncy = ty * h + cy
        nw = jnp.exp(tw) * w
        nh = jnp.exp(th) * h
        boxes_ref[0] = ncy - 0.5 * nh
        boxes_ref[1] = ncx - 0.5 * nw
        boxes_ref[2] = ncy + 0.5 * nh
        boxes_ref[3] = ncx + 0.5 * nw

    valid_ref[...] = scores_ref[0]

    row_i = jax.lax.broadcasted_iota(jnp.int32, (_ROWS, _LANES), 0)
    col_i = jax.lax.broadcasted_iota(jnp.int32, (_ROWS, _LANES), 1)
    flat_i = row_i * _LANES + col_i
    lane_i = jax.lax.broadcasted_iota(jnp.int32, (1, _LANES), 1)

    big = jnp.int32(2 ** 30)
    deg = jnp.float32(1e30)

    def pop(_, carry):
        count, s_s, s_y1, s_x1, s_y2, s_x2 = carry
        v = valid_ref[...]
        m = jnp.max(v)
        j = jnp.min(jnp.where(v == m, flat_i, big))
        r = j // _LANES
        col = j - r * _LANES
        hit = lane_i == col

        def comp(k):
            rowv = boxes_ref[k, pl.ds(r, 1), :]
            return jnp.sum(jnp.where(hit, rowv, 0.0))

        by1, bx1, by2, bx2 = comp(0), comp(1), comp(2), comp(3)

        yy1 = jnp.maximum(by1, s_y1)
        xx1 = jnp.maximum(bx1, s_x1)
        yy2 = jnp.minimum(by2, s_y2)
        xx2 = jnp.minimum(bx2, s_x2)
        inter = jnp.maximum(yy2 - yy1, 0.0) * jnp.maximum(xx2 - xx1, 0.0)
        area1 = (by2 - by1) * (bx2 - bx1)
        areas = (s_y2 - s_y1) * (s_x2 - s_x1)
        iou = inter / jnp.maximum(area1 + areas - inter, 1e-9)
        overlapped = jnp.max(iou) > _IOU_T

        alive = m > _NEG
        keep = alive & jnp.logical_not(overlapped) & (count < _K)

        vrow = valid_ref[pl.ds(r, 1), :]
        valid_ref[pl.ds(r, 1), :] = jnp.where(hit, _NEG, vrow)

        slot = (lane_i == count) & keep
        s_s = jnp.where(slot, m, s_s)
        s_y1 = jnp.where(slot, by1, s_y1)
        s_x1 = jnp.where(slot, bx1, s_x1)
        s_y2 = jnp.where(slot, by2, s_y2)
        s_x2 = jnp.where(slot, bx2, s_x2)
        count = count + keep.astype(jnp.int32)
        return count, s_s, s_y1, s_x1, s_y2, s_x2

    neg = jnp.full((1, _LANES), _NEG, jnp.float32)
    degv = jnp.full((1, _LANES), deg, jnp.float32)
    count, s_s, s_y1, s_x1, s_y2, s_x2 = jax.lax.fori_loop(
        0, _POPS, pop, (jnp.int32(0), neg, degv, degv, degv, degv))

    b0y1 = boxes_ref[0, 0:1, 0:1]
    b0x1 = boxes_ref[1, 0:1, 0:1]
    b0y2 = boxes_ref[2, 0:1, 0:1]
    b0x2 = boxes_ref[3, 0:1, 0:1]
    filled = lane_i < count
    sel_s_ref[0] = s_s
    sel_b_ref[0] = jnp.concatenate([
        jnp.where(filled, s_y1, b0y1),
        jnp.where(filled, s_x1, b0x1),
        jnp.where(filled, s_y2, b0y2),
        jnp.where(filled, s_x2, b0x2),
    ], axis=0)


_MROWS = 8
_MCOLS = _C * _LANES // _MROWS


def _topk_body(scores_ref, planes_ref, boxes_ref, cls_ref, valid_ref):
    valid_ref[...] = scores_ref[...]
    row_i = jax.lax.broadcasted_iota(jnp.int32, (_MROWS, _MCOLS), 0)
    col_i = jax.lax.broadcasted_iota(jnp.int32, (_MROWS, _MCOLS), 1)
    flat_i = row_i * _MCOLS + col_i
    lane_i = jax.lax.broadcasted_iota(jnp.int32, (1, _MCOLS), 1)
    big = jnp.int32(2 ** 30)

    def step(i, _):
        v = valid_ref[...]
        m = jnp.max(v)
        j = jnp.min(jnp.where(v == m, flat_i, big))
        r = j // _MCOLS
        col = j - r * _MCOLS
        hit = lane_i == col

        vrow = valid_ref[pl.ds(r, 1), :]
        valid_ref[pl.ds(r, 1), :] = jnp.where(hit, _NEG, vrow)

        def comp(k):
            rowv = planes_ref[k, pl.ds(r, 1), :]
            return jnp.sum(jnp.where(hit, rowv, 0.0)).reshape(1, 1)

        boxes_ref[pl.ds(i, 1), :] = jnp.concatenate(
            [comp(0), comp(1), comp(2), comp(3)], axis=1)
        cls_id = (j // _LANES).reshape(1, 1)
        cls_ref[pl.ds(i, 1), :] = jnp.broadcast_to(cls_id, (1, 4))
        return 0

    jax.lax.fori_loop(0, _OUT, step, 0)


def kernel(roi_bboxes_txtytwth, roi_score, rpn_proposals_bboxes):
    npad = _ROWS * _LANES - _N

    props = jnp.pad(rpn_proposals_bboxes, ((0, npad), (0, 0)))
    props = props.T.reshape(4, _ROWS, _LANES)
    deltas = jnp.pad(roi_bboxes_txtytwth, ((0, npad), (0, 0)))
    deltas = deltas.T.reshape(4, _ROWS, _LANES)
    scores = jnp.pad(roi_score.T, ((0, 0), (0, npad)),
                     constant_values=_NEG).reshape(_C, _ROWS, _LANES)

    sel_s, sel_b = pl.pallas_call(
        _nms_body,
        grid=(_C,),
        in_specs=[
            pl.BlockSpec((4, _ROWS, _LANES), lambda c: (0, 0, 0)),
            pl.BlockSpec((4, _ROWS, _LANES), lambda c: (0, 0, 0)),
            pl.BlockSpec((1, _ROWS, _LANES), lambda c: (c, 0, 0)),
        ],
        out_specs=[
            pl.BlockSpec((1, 1, _LANES), lambda c: (c, 0, 0)),
            pl.BlockSpec((1, 4, _LANES), lambda c: (c, 0, 0)),
        ],
        out_shape=[
            jax.ShapeDtypeStruct((_C, 1, _LANES), jnp.float32),
            jax.ShapeDtypeStruct((_C, 4, _LANES), jnp.float32),
        ],
        scratch_shapes=[
            pltpu.VMEM((4, _ROWS, _LANES), jnp.float32),
            pltpu.VMEM((_ROWS, _LANES), jnp.float32),
        ],
    )(props, deltas, scores)

    merged_s = sel_s.reshape(_MROWS, _MCOLS)
    merged_b = sel_b.transpose(1, 0, 2).reshape(4, _MROWS, _MCOLS)

    boxes_out, cls_out = pl.pallas_call(
        _topk_body,
        out_shape=[
            jax.ShapeDtypeStruct((_OUT + 4, 4), jnp.float32),
            jax.ShapeDtypeStruct((_OUT + 4, 4), jnp.int32),
        ],
        scratch_shapes=[pltpu.VMEM((_MROWS, _MCOLS), jnp.float32)],
    )(merged_s, merged_b)

    return boxes_out[:_OUT], cls_out[:_OUT, 0]


# speedup vs baseline: 1.0818x; 1.0818x over previous
import jax
import jax.numpy as jnp
from jax.experimental import pallas as pl
from jax.experimental.pallas import tpu as pltpu

_C = 80
_K = 100
_OUT = 300
_IOU_T = 0.7
_N = 20000
_ROWS = 160
_LANES = 128
_CHUNK = 8
_NCHUNK = _ROWS // _CHUNK
_NEG = float("-inf")


def _nms_body(props_ref, deltas_ref, scores_ref, sel_s_ref, sel_b_ref,
              boxes_ref, valid_ref):
    c = pl.program_id(0)

    @pl.when(c == 0)
    def _decode():
        py1 = props_ref[0]
        px1 = props_ref[1]
        py2 = props_ref[2]
        px2 = props_ref[3]
        h = py2 - py1
        w = px2 - px1
        cy = py1 + 0.5 * h
        cx = px1 + 0.5 * w
        tx = deltas_ref[0]
        ty = deltas_ref[1]
        tw = deltas_ref[2]
        th = deltas_ref[3]
        ncx = tx * w + cx
        ncy = ty * h + cy
        nw = jnp.exp(tw) * w
        nh = jnp.exp(th) * h
        boxes_ref[0] = ncy - 0.5 * nh
        boxes_ref[1] = ncx - 0.5 * nw
        boxes_ref[2] = ncy + 0.5 * nh
        boxes_ref[3] = ncx + 0.5 * nw

    valid_ref[...] = scores_ref[0]

    row8 = jax.lax.broadcasted_iota(jnp.int32, (_CHUNK, _LANES), 0)
    col8 = jax.lax.broadcasted_iota(jnp.int32, (_CHUNK, _LANES), 1)
    flat8 = row8 * _LANES + col8
    lane_i = jax.lax.broadcasted_iota(jnp.int32, (1, _LANES), 1)

    big = jnp.int32(2 ** 30)
    deg = jnp.float32(1e30)

    cm = jnp.full((1, _LANES), _NEG, jnp.float32)
    for g in range(_NCHUNK):
        mg = jnp.max(valid_ref[g * _CHUNK:(g + 1) * _CHUNK, :])
        cm = jnp.where(lane_i == g, mg, cm)

    def cond(carry):
        count, cm, *_ = carry
        return (count < _K) & (jnp.max(cm) > _NEG)

    def pop(carry):
        count, cm, s_s, s_y1, s_x1, s_y2, s_x2 = carry
        m = jnp.max(cm)
        ch = jnp.min(jnp.where(cm == m, lane_i, big))
        base = ch * _CHUNK
        v8 = valid_ref[pl.ds(base, _CHUNK), :]
        j8 = jnp.min(jnp.where(v8 == m, flat8, big))
        r8 = j8 // _LANES
        col = j8 - r8 * _LANES
        hit = lane_i == col
        r = base + r8

        def comp(k):
            rowv = boxes_ref[k, pl.ds(r, 1), :]
            return jnp.sum(jnp.where(hit, rowv, 0.0))

        by1, bx1, by2, bx2 = comp(0), comp(1), comp(2), comp(3)

        yy1 = jnp.maximum(by1, s_y1)
        xx1 = jnp.maximum(bx1, s_x1)
        yy2 = jnp.minimum(by2, s_y2)
        xx2 = jnp.minimum(bx2, s_x2)
        inter = jnp.maximum(yy2 - yy1, 0.0) * jnp.maximum(xx2 - xx1, 0.0)
        area1 = (by2 - by1) * (bx2 - bx1)
        areas = (s_y2 - s_y1) * (s_x2 - s_x1)
        iou = inter / jnp.maximum(area1 + areas - inter, 1e-9)
        keep = jnp.logical_not(jnp.max(iou) > _IOU_T)

        hit8 = (row8 == r8) & (col8 == col)
        v8n = jnp.where(hit8, _NEG, v8)
        valid_ref[pl.ds(base, _CHUNK), :] = v8n
        cm = jnp.where(lane_i == ch, jnp.max(v8n), cm)

        slot = (lane_i == count) & keep
        s_s = jnp.where(slot, m, s_s)
        s_y1 = jnp.where(slot, by1, s_y1)
        s_x1 = jnp.where(slot, bx1, s_x1)
        s_y2 = jnp.where(slot, by2, s_y2)
        s_x2 = jnp.where(slot, bx2, s_x2)
        count = count + keep.astype(jnp.int32)
        return count, cm, s_s, s_y1, s_x1, s_y2, s_x2

    neg = jnp.full((1, _LANES), _NEG, jnp.float32)
    degv = jnp.full((1, _LANES), deg, jnp.float32)
    count, _, s_s, s_y1, s_x1, s_y2, s_x2 = jax.lax.while_loop(
        cond, pop, (jnp.int32(0), cm, neg, degv, degv, degv, degv))

    b0y1 = boxes_ref[0, 0:1, 0:1]
    b0x1 = boxes_ref[1, 0:1, 0:1]
    b0y2 = boxes_ref[2, 0:1, 0:1]
    b0x2 = boxes_ref[3, 0:1, 0:1]
    filled = lane_i < count
    sel_s_ref[0] = s_s
    sel_b_ref[0] = jnp.concatenate([
        jnp.where(filled, s_y1, b0y1),
        jnp.where(filled, s_x1, b0x1),
        jnp.where(filled, s_y2, b0y2),
        jnp.where(filled, s_x2, b0x2),
    ], axis=0)


_MROWS = 8
_MCOLS = _C * _LANES // _MROWS


def _topk_body(scores_ref, planes_ref, boxes_ref, cls_ref, valid_ref):
    valid_ref[...] = scores_ref[...]
    row_i = jax.lax.broadcasted_iota(jnp.int32, (_MROWS, _MCOLS), 0)
    col_i = jax.lax.broadcasted_iota(jnp.int32, (_MROWS, _MCOLS), 1)
    flat_i = row_i * _MCOLS + col_i
    lane_i = jax.lax.broadcasted_iota(jnp.int32, (1, _MCOLS), 1)
    big = jnp.int32(2 ** 30)

    def step(i, _):
        v = valid_ref[...]
        m = jnp.max(v)
        j = jnp.min(jnp.where(v == m, flat_i, big))
        r = j // _MCOLS
        col = j - r * _MCOLS
        hit = lane_i == col

        vrow = valid_ref[pl.ds(r, 1), :]
        valid_ref[pl.ds(r, 1), :] = jnp.where(hit, _NEG, vrow)

        def comp(k):
            rowv = planes_ref[k, pl.ds(r, 1), :]
            return jnp.sum(jnp.where(hit, rowv, 0.0)).reshape(1, 1)

        boxes_ref[pl.ds(i, 1), :] = jnp.concatenate(
            [comp(0), comp(1), comp(2), comp(3)], axis=1)
        cls_id = (j // _LANES).reshape(1, 1)
        cls_ref[pl.ds(i, 1), :] = jnp.broadcast_to(cls_id, (1, 4))
        return 0

    jax.lax.fori_loop(0, _OUT, step, 0)


def kernel(roi_bboxes_txtytwth, roi_score, rpn_proposals_bboxes):
    npad = _ROWS * _LANES - _N

    props = jnp.pad(rpn_proposals_bboxes, ((0, npad), (0, 0)))
    props = props.T.reshape(4, _ROWS, _LANES)
    deltas = jnp.pad(roi_bboxes_txtytwth, ((0, npad), (0, 0)))
    deltas = deltas.T.reshape(4, _ROWS, _LANES)
    scores = jnp.pad(roi_score.T, ((0, 0), (0, npad)),
                     constant_values=_NEG).reshape(_C, _ROWS, _LANES)

    sel_s, sel_b = pl.pallas_call(
        _nms_body,
        grid=(_C,),
        in_specs=[
            pl.BlockSpec((4, _ROWS, _LANES), lambda c: (0, 0, 0)),
            pl.BlockSpec((4, _ROWS, _LANES), lambda c: (0, 0, 0)),
            pl.BlockSpec((1, _ROWS, _LANES), lambda c: (c, 0, 0)),
        ],
        out_specs=[
            pl.BlockSpec((1, 1, _LANES), lambda c: (c, 0, 0)),
            pl.BlockSpec((1, 4, _LANES), lambda c: (c, 0, 0)),
        ],
        out_shape=[
            jax.ShapeDtypeStruct((_C, 1, _LANES), jnp.float32),
            jax.ShapeDtypeStruct((_C, 4, _LANES), jnp.float32),
        ],
        scratch_shapes=[
            pltpu.VMEM((4, _ROWS, _LANES), jnp.float32),
            pltpu.VMEM((_ROWS, _LANES), jnp.float32),
        ],
    )(props, deltas, scores)

    merged_s = sel_s.reshape(_MROWS, _MCOLS)
    merged_b = sel_b.transpose(1, 0, 2).reshape(4, _MROWS, _MCOLS)

    boxes_out, cls_out = pl.pallas_call(
        _topk_body,
        out_shape=[
            jax.ShapeDtypeStruct((_OUT + 4, 4), jnp.float32),
            jax.ShapeDtypeStruct((_OUT + 4, 4), jnp.int32),
        ],
        scratch_shapes=[pltpu.VMEM((_MROWS, _MCOLS), jnp.float32)],
    )(merged_s, merged_b)

    return boxes_out[:_OUT], cls_out[:_OUT, 0]


# speedup vs baseline: 1.3275x; 1.2272x over previous
import jax
import jax.numpy as jnp
from jax.experimental import pallas as pl
from jax.experimental.pallas import tpu as pltpu

_C = 80
_K = 100
_OUT = 300
_IOU_T = 0.7
_N = 20000
_ROWS = 160
_LANES = 128
_CHUNK = 8
_NCHUNK = _ROWS // _CHUNK
_NEG = float("-inf")


_G = 4


def _nms_body(props_ref, deltas_ref, scores_ref, sel_s_ref, sel_b_ref,
              boxes_ref, valid_ref):
    c = pl.program_id(0)

    @pl.when(c == 0)
    def _decode():
        py1 = props_ref[0]
        px1 = props_ref[1]
        py2 = props_ref[2]
        px2 = props_ref[3]
        h = py2 - py1
        w = px2 - px1
        cy = py1 + 0.5 * h
        cx = px1 + 0.5 * w
        tx = deltas_ref[0]
        ty = deltas_ref[1]
        tw = deltas_ref[2]
        th = deltas_ref[3]
        ncx = tx * w + cx
        ncy = ty * h + cy
        nw = jnp.exp(tw) * w
        nh = jnp.exp(th) * h
        boxes_ref[0] = ncy - 0.5 * nh
        boxes_ref[1] = ncx - 0.5 * nw
        boxes_ref[2] = ncy + 0.5 * nh
        boxes_ref[3] = ncx + 0.5 * nw

    valid_ref[...] = scores_ref[...]

    row8 = jax.lax.broadcasted_iota(jnp.int32, (_CHUNK, _LANES), 0)
    col8 = jax.lax.broadcasted_iota(jnp.int32, (_CHUNK, _LANES), 1)
    flat8 = row8 * _LANES + col8
    lane_i = jax.lax.broadcasted_iota(jnp.int32, (1, _LANES), 1)

    big = jnp.int32(2 ** 30)
    deg = jnp.float32(1e30)

    cm0 = []
    for k in range(_G):
        cm = jnp.full((1, _LANES), _NEG, jnp.float32)
        for g in range(_NCHUNK):
            mg = jnp.max(valid_ref[k, g * _CHUNK:(g + 1) * _CHUNK, :])
            cm = jnp.where(lane_i == g, mg, cm)
        cm0.append(cm)

    def cond(carry):
        counts, cms = carry[0], carry[1]
        act = jnp.bool_(False)
        for k in range(_G):
            act |= (counts[k] < _K) & (jnp.max(cms[k]) > _NEG)
        return act

    def pop(carry):
        counts, cms, s_s, s_y1, s_x1, s_y2, s_x2 = carry
        ncounts, ncms = [], []
        ns_s, ns_y1, ns_x1, ns_y2, ns_x2 = [], [], [], [], []
        for k in range(_G):
            count, cm = counts[k], cms[k]
            m = jnp.max(cm)
            active = (count < _K) & (m > _NEG)
            ch = jnp.min(jnp.where(cm == m, lane_i, big))
            base = ch * _CHUNK
            v8 = valid_ref[k, pl.ds(base, _CHUNK), :]
            j8 = jnp.min(jnp.where(v8 == m, flat8, big))
            r8 = j8 // _LANES
            col = j8 - r8 * _LANES
            hit = lane_i == col
            r = base + r8

            def comp(q):
                rowv = boxes_ref[q, pl.ds(r, 1), :]
                return jnp.sum(jnp.where(hit, rowv, 0.0))

            by1, bx1, by2, bx2 = comp(0), comp(1), comp(2), comp(3)

            yy1 = jnp.maximum(by1, s_y1[k])
            xx1 = jnp.maximum(bx1, s_x1[k])
            yy2 = jnp.minimum(by2, s_y2[k])
            xx2 = jnp.minimum(bx2, s_x2[k])
            inter = jnp.maximum(yy2 - yy1, 0.0) * jnp.maximum(xx2 - xx1, 0.0)
            area1 = (by2 - by1) * (bx2 - bx1)
            areas = (s_y2[k] - s_y1[k]) * (s_x2[k] - s_x1[k])
            iou = inter / jnp.maximum(area1 + areas - inter, 1e-9)
            keep = active & jnp.logical_not(jnp.max(iou) > _IOU_T)

            hit8 = (row8 == r8) & (col8 == col)
            v8n = jnp.where(hit8, _NEG, v8)
            valid_ref[k, pl.ds(base, _CHUNK), :] = v8n
            ncms.append(jnp.where(lane_i == ch, jnp.max(v8n), cm))

            slot = (lane_i == count) & keep
            ns_s.append(jnp.where(slot, m, s_s[k]))
            ns_y1.append(jnp.where(slot, by1, s_y1[k]))
            ns_x1.append(jnp.where(slot, bx1, s_x1[k]))
            ns_y2.append(jnp.where(slot, by2, s_y2[k]))
            ns_x2.append(jnp.where(slot, bx2, s_x2[k]))
            ncounts.append(count + keep.astype(jnp.int32))
        return (tuple(ncounts), tuple(ncms), tuple(ns_s), tuple(ns_y1),
                tuple(ns_x1), tuple(ns_y2), tuple(ns_x2))

    neg = jnp.full((1, _LANES), _NEG, jnp.float32)
    degv = jnp.full((1, _LANES), deg, jnp.float32)
    zero = jnp.int32(0)
    init = (tuple(zero for _ in range(_G)), tuple(cm0),
            tuple(neg for _ in range(_G)), tuple(degv for _ in range(_G)),
            tuple(degv for _ in range(_G)), tuple(degv for _ in range(_G)),
            tuple(degv for _ in range(_G)))
    counts, _, s_s, s_y1, s_x1, s_y2, s_x2 = jax.lax.while_loop(
        cond, pop, init)

    b0y1 = boxes_ref[0, 0:1, 0:1]
    b0x1 = boxes_ref[1, 0:1, 0:1]
    b0y2 = boxes_ref[2, 0:1, 0:1]
    b0x2 = boxes_ref[3, 0:1, 0:1]
    for k in range(_G):
        filled = lane_i < counts[k]
        sel_s_ref[k] = s_s[k]
        sel_b_ref[k] = jnp.concatenate([
            jnp.where(filled, s_y1[k], b0y1),
            jnp.where(filled, s_x1[k], b0x1),
            jnp.where(filled, s_y2[k], b0y2),
            jnp.where(filled, s_x2[k], b0x2),
        ], axis=0)


_MROWS = 8
_MCOLS = _C * _LANES // _MROWS


def _topk_body(scores_ref, planes_ref, boxes_ref, cls_ref, valid_ref):
    valid_ref[...] = scores_ref[...]
    row_i = jax.lax.broadcasted_iota(jnp.int32, (_MROWS, _MCOLS), 0)
    col_i = jax.lax.broadcasted_iota(jnp.int32, (_MROWS, _MCOLS), 1)
    flat_i = row_i * _MCOLS + col_i
    lane_i = jax.lax.broadcasted_iota(jnp.int32, (1, _MCOLS), 1)
    big = jnp.int32(2 ** 30)

    def step(i, _):
        v = valid_ref[...]
        m = jnp.max(v)
        j = jnp.min(jnp.where(v == m, flat_i, big))
        r = j // _MCOLS
        col = j - r * _MCOLS
        hit = lane_i == col

        vrow = valid_ref[pl.ds(r, 1), :]
        valid_ref[pl.ds(r, 1), :] = jnp.where(hit, _NEG, vrow)

        def comp(k):
            rowv = planes_ref[k, pl.ds(r, 1), :]
            return jnp.sum(jnp.where(hit, rowv, 0.0)).reshape(1, 1)

        boxes_ref[pl.ds(i, 1), :] = jnp.concatenate(
            [comp(0), comp(1), comp(2), comp(3)], axis=1)
        cls_id = (j // _LANES).reshape(1, 1)
        cls_ref[pl.ds(i, 1), :] = jnp.broadcast_to(cls_id, (1, 4))
        return 0

    jax.lax.fori_loop(0, _OUT, step, 0)


def kernel(roi_bboxes_txtytwth, roi_score, rpn_proposals_bboxes):
    npad = _ROWS * _LANES - _N

    props = jnp.pad(rpn_proposals_bboxes, ((0, npad), (0, 0)))
    props = props.T.reshape(4, _ROWS, _LANES)
    deltas = jnp.pad(roi_bboxes_txtytwth, ((0, npad), (0, 0)))
    deltas = deltas.T.reshape(4, _ROWS, _LANES)
    scores = jnp.pad(roi_score.T, ((0, 0), (0, npad)),
                     constant_values=_NEG).reshape(_C, _ROWS, _LANES)

    sel_s, sel_b = pl.pallas_call(
        _nms_body,
        grid=(_C // _G,),
        in_specs=[
            pl.BlockSpec((4, _ROWS, _LANES), lambda c: (0, 0, 0)),
            pl.BlockSpec((4, _ROWS, _LANES), lambda c: (0, 0, 0)),
            pl.BlockSpec((_G, _ROWS, _LANES), lambda c: (c, 0, 0)),
        ],
        out_specs=[
            pl.BlockSpec((_G, 1, _LANES), lambda c: (c, 0, 0)),
            pl.BlockSpec((_G, 4, _LANES), lambda c: (c, 0, 0)),
        ],
        out_shape=[
            jax.ShapeDtypeStruct((_C, 1, _LANES), jnp.float32),
            jax.ShapeDtypeStruct((_C, 4, _LANES), jnp.float32),
        ],
        scratch_shapes=[
            pltpu.VMEM((4, _ROWS, _LANES), jnp.float32),
            pltpu.VMEM((_G, _ROWS, _LANES), jnp.float32),
        ],
    )(props, deltas, scores)

    merged_s = sel_s.reshape(_MROWS, _MCOLS)
    merged_b = sel_b.transpose(1, 0, 2).reshape(4, _MROWS, _MCOLS)

    boxes_out, cls_out = pl.pallas_call(
        _topk_body,
        out_shape=[
            jax.ShapeDtypeStruct((_OUT + 4, 4), jnp.float32),
            jax.ShapeDtypeStruct((_OUT + 4, 4), jnp.int32),
        ],
        scratch_shapes=[pltpu.VMEM((_MROWS, _MCOLS), jnp.float32)],
    )(merged_s, merged_b)

    return boxes_out[:_OUT], cls_out[:_OUT, 0]


# speedup vs baseline: 1.3798x; 1.0393x over previous
import jax
import jax.numpy as jnp
from jax.experimental import pallas as pl
from jax.experimental.pallas import tpu as pltpu

_C = 80
_K = 100
_OUT = 300
_IOU_T = 0.7
_N = 20000
_ROWS = 160
_LANES = 128
_CHUNK = 8
_NCHUNK = _ROWS // _CHUNK
_NEG = float("-inf")


_G = 8


def _nms_body(props_ref, deltas_ref, scores_ref, sel_s_ref, sel_b_ref,
              boxes_ref, valid_ref):
    c = pl.program_id(0)

    @pl.when(c == 0)
    def _decode():
        py1 = props_ref[0]
        px1 = props_ref[1]
        py2 = props_ref[2]
        px2 = props_ref[3]
        h = py2 - py1
        w = px2 - px1
        cy = py1 + 0.5 * h
        cx = px1 + 0.5 * w
        tx = deltas_ref[0]
        ty = deltas_ref[1]
        tw = deltas_ref[2]
        th = deltas_ref[3]
        ncx = tx * w + cx
        ncy = ty * h + cy
        nw = jnp.exp(tw) * w
        nh = jnp.exp(th) * h
        boxes_ref[0] = ncy - 0.5 * nh
        boxes_ref[1] = ncx - 0.5 * nw
        boxes_ref[2] = ncy + 0.5 * nh
        boxes_ref[3] = ncx + 0.5 * nw

    valid_ref[...] = scores_ref[...]

    row8 = jax.lax.broadcasted_iota(jnp.int32, (_CHUNK, _LANES), 0)
    col8 = jax.lax.broadcasted_iota(jnp.int32, (_CHUNK, _LANES), 1)
    flat8 = row8 * _LANES + col8
    lane_i = jax.lax.broadcasted_iota(jnp.int32, (1, _LANES), 1)

    big = jnp.int32(2 ** 30)
    deg = jnp.float32(1e30)

    cm0 = []
    for k in range(_G):
        cm = jnp.full((1, _LANES), _NEG, jnp.float32)
        for g in range(_NCHUNK):
            mg = jnp.max(valid_ref[k, g * _CHUNK:(g + 1) * _CHUNK, :])
            cm = jnp.where(lane_i == g, mg, cm)
        cm0.append(cm)

    def cond(carry):
        counts, cms = carry[0], carry[1]
        act = jnp.bool_(False)
        for k in range(_G):
            act |= (counts[k] < _K) & (jnp.max(cms[k]) > _NEG)
        return act

    def pop(carry):
        counts, cms, s_s, s_y1, s_x1, s_y2, s_x2 = carry
        ncounts, ncms = [], []
        ns_s, ns_y1, ns_x1, ns_y2, ns_x2 = [], [], [], [], []
        for k in range(_G):
            count, cm = counts[k], cms[k]
            m = jnp.max(cm)
            active = (count < _K) & (m > _NEG)
            ch = jnp.min(jnp.where(cm == m, lane_i, big))
            base = ch * _CHUNK
            v8 = valid_ref[k, pl.ds(base, _CHUNK), :]
            j8 = jnp.min(jnp.where(v8 == m, flat8, big))
            r8 = j8 // _LANES
            col = j8 - r8 * _LANES
            hit = lane_i == col
            r = base + r8

            def comp(q):
                rowv = boxes_ref[q, pl.ds(r, 1), :]
                return jnp.sum(jnp.where(hit, rowv, 0.0))

            by1, bx1, by2, bx2 = comp(0), comp(1), comp(2), comp(3)

            yy1 = jnp.maximum(by1, s_y1[k])
            xx1 = jnp.maximum(bx1, s_x1[k])
            yy2 = jnp.minimum(by2, s_y2[k])
            xx2 = jnp.minimum(bx2, s_x2[k])
            inter = jnp.maximum(yy2 - yy1, 0.0) * jnp.maximum(xx2 - xx1, 0.0)
            area1 = (by2 - by1) * (bx2 - bx1)
            areas = (s_y2[k] - s_y1[k]) * (s_x2[k] - s_x1[k])
            iou = inter / jnp.maximum(area1 + areas - inter, 1e-9)
            keep = active & jnp.logical_not(jnp.max(iou) > _IOU_T)

            hit8 = (row8 == r8) & (col8 == col)
            v8n = jnp.where(hit8, _NEG, v8)
            valid_ref[k, pl.ds(base, _CHUNK), :] = v8n
            ncms.append(jnp.where(lane_i == ch, jnp.max(v8n), cm))

            slot = (lane_i == count) & keep
            ns_s.append(jnp.where(slot, m, s_s[k]))
            ns_y1.append(jnp.where(slot, by1, s_y1[k]))
            ns_x1.append(jnp.where(slot, bx1, s_x1[k]))
            ns_y2.append(jnp.where(slot, by2, s_y2[k]))
            ns_x2.append(jnp.where(slot, bx2, s_x2[k]))
            ncounts.append(count + keep.astype(jnp.int32))
        return (tuple(ncounts), tuple(ncms), tuple(ns_s), tuple(ns_y1),
                tuple(ns_x1), tuple(ns_y2), tuple(ns_x2))

    neg = jnp.full((1, _LANES), _NEG, jnp.float32)
    degv = jnp.full((1, _LANES), deg, jnp.float32)
    zero = jnp.int32(0)
    init = (tuple(zero for _ in range(_G)), tuple(cm0),
            tuple(neg for _ in range(_G)), tuple(degv for _ in range(_G)),
            tuple(degv for _ in range(_G)), tuple(degv for _ in range(_G)),
            tuple(degv for _ in range(_G)))
    counts, _, s_s, s_y1, s_x1, s_y2, s_x2 = jax.lax.while_loop(
        cond, pop, init)

    b0y1 = boxes_ref[0, 0:1, 0:1]
    b0x1 = boxes_ref[1, 0:1, 0:1]
    b0y2 = boxes_ref[2, 0:1, 0:1]
    b0x2 = boxes_ref[3, 0:1, 0:1]
    for k in range(_G):
        filled = lane_i < counts[k]
        sel_s_ref[k] = s_s[k]
        sel_b_ref[k] = jnp.concatenate([
            jnp.where(filled, s_y1[k], b0y1),
            jnp.where(filled, s_x1[k], b0x1),
            jnp.where(filled, s_y2[k], b0y2),
            jnp.where(filled, s_x2[k], b0x2),
        ], axis=0)


_MROWS = 8
_MCOLS = _C * _LANES // _MROWS


def _topk_body(scores_ref, planes_ref, boxes_ref, cls_ref, valid_ref):
    valid_ref[...] = scores_ref[...]
    row_i = jax.lax.broadcasted_iota(jnp.int32, (_MROWS, _MCOLS), 0)
    col_i = jax.lax.broadcasted_iota(jnp.int32, (_MROWS, _MCOLS), 1)
    flat_i = row_i * _MCOLS + col_i
    lane_i = jax.lax.broadcasted_iota(jnp.int32, (1, _MCOLS), 1)
    big = jnp.int32(2 ** 30)

    def step(i, _):
        v = valid_ref[...]
        m = jnp.max(v)
        j = jnp.min(jnp.where(v == m, flat_i, big))
        r = j // _MCOLS
        col = j - r * _MCOLS
        hit = lane_i == col

        vrow = valid_ref[pl.ds(r, 1), :]
        valid_ref[pl.ds(r, 1), :] = jnp.where(hit, _NEG, vrow)

        def comp(k):
            rowv = planes_ref[k, pl.ds(r, 1), :]
            return jnp.sum(jnp.where(hit, rowv, 0.0)).reshape(1, 1)

        boxes_ref[pl.ds(i, 1), :] = jnp.concatenate(
            [comp(0), comp(1), comp(2), comp(3)], axis=1)
        cls_id = (j // _LANES).reshape(1, 1)
        cls_ref[pl.ds(i, 1), :] = jnp.broadcast_to(cls_id, (1, 4))
        return 0

    jax.lax.fori_loop(0, _OUT, step, 0)


def kernel(roi_bboxes_txtytwth, roi_score, rpn_proposals_bboxes):
    npad = _ROWS * _LANES - _N

    props = jnp.pad(rpn_proposals_bboxes, ((0, npad), (0, 0)))
    props = props.T.reshape(4, _ROWS, _LANES)
    deltas = jnp.pad(roi_bboxes_txtytwth, ((0, npad), (0, 0)))
    deltas = deltas.T.reshape(4, _ROWS, _LANES)
    scores = jnp.pad(roi_score.T, ((0, 0), (0, npad)),
                     constant_values=_NEG).reshape(_C, _ROWS, _LANES)

    sel_s, sel_b = pl.pallas_call(
        _nms_body,
        grid=(_C // _G,),
        in_specs=[
            pl.BlockSpec((4, _ROWS, _LANES), lambda c: (0, 0, 0)),
            pl.BlockSpec((4, _ROWS, _LANES), lambda c: (0, 0, 0)),
            pl.BlockSpec((_G, _ROWS, _LANES), lambda c: (c, 0, 0)),
        ],
        out_specs=[
            pl.BlockSpec((_G, 1, _LANES), lambda c: (c, 0, 0)),
            pl.BlockSpec((_G, 4, _LANES), lambda c: (c, 0, 0)),
        ],
        out_shape=[
            jax.ShapeDtypeStruct((_C, 1, _LANES), jnp.float32),
            jax.ShapeDtypeStruct((_C, 4, _LANES), jnp.float32),
        ],
        scratch_shapes=[
            pltpu.VMEM((4, _ROWS, _LANES), jnp.float32),
            pltpu.VMEM((_G, _ROWS, _LANES), jnp.float32),
        ],
    )(props, deltas, scores)

    merged_s = sel_s.reshape(_MROWS, _MCOLS)
    merged_b = sel_b.transpose(1, 0, 2).reshape(4, _MROWS, _MCOLS)

    boxes_out, cls_out = pl.pallas_call(
        _topk_body,
        out_shape=[
            jax.ShapeDtypeStruct((_OUT + 4, 4), jnp.float32),
            jax.ShapeDtypeStruct((_OUT + 4, 4), jnp.int32),
        ],
        scratch_shapes=[pltpu.VMEM((_MROWS, _MCOLS), jnp.float32)],
    )(merged_s, merged_b)

    return boxes_out[:_OUT], cls_out[:_OUT, 0]


# speedup vs baseline: 1.3801x; 1.0003x over previous
import jax
import jax.numpy as jnp
from jax.experimental import pallas as pl
from jax.experimental.pallas import tpu as pltpu

_C = 80
_K = 100
_OUT = 300
_IOU_T = 0.7
_N = 20000
_ROWS = 160
_LANES = 128
_CHUNK = 8
_NCHUNK = _ROWS // _CHUNK
_NEG = float("-inf")


_G = 8


def _nms_body(props_ref, deltas_ref, scores_ref, sel_s_ref, sel_b_ref,
              boxes_ref, *valid_refs):
    c = pl.program_id(0)

    @pl.when(c == 0)
    def _decode():
        py1 = props_ref[0]
        px1 = props_ref[1]
        py2 = props_ref[2]
        px2 = props_ref[3]
        h = py2 - py1
        w = px2 - px1
        cy = py1 + 0.5 * h
        cx = px1 + 0.5 * w
        tx = deltas_ref[0]
        ty = deltas_ref[1]
        tw = deltas_ref[2]
        th = deltas_ref[3]
        ncx = tx * w + cx
        ncy = ty * h + cy
        nw = jnp.exp(tw) * w
        nh = jnp.exp(th) * h
        boxes_ref[0] = ncy - 0.5 * nh
        boxes_ref[1] = ncx - 0.5 * nw
        boxes_ref[2] = ncy + 0.5 * nh
        boxes_ref[3] = ncx + 0.5 * nw

    for k in range(_G):
        valid_refs[k][...] = scores_ref[k]

    row8 = jax.lax.broadcasted_iota(jnp.int32, (_CHUNK, _LANES), 0)
    col8 = jax.lax.broadcasted_iota(jnp.int32, (_CHUNK, _LANES), 1)
    flat8 = row8 * _LANES + col8
    lane_i = jax.lax.broadcasted_iota(jnp.int32, (1, _LANES), 1)

    big = jnp.int32(2 ** 30)
    deg = jnp.float32(1e30)

    cm0 = []
    for k in range(_G):
        cm = jnp.full((1, _LANES), _NEG, jnp.float32)
        for g in range(_NCHUNK):
            mg = jnp.max(valid_refs[k][g * _CHUNK:(g + 1) * _CHUNK, :])
            cm = jnp.where(lane_i == g, mg, cm)
        cm0.append(cm)

    def cond(carry):
        counts, cms = carry[0], carry[1]
        act = jnp.bool_(False)
        for k in range(_G):
            act |= (counts[k] < _K) & (jnp.max(cms[k]) > _NEG)
        return act

    def pop(carry):
        counts, cms, s_s, s_y1, s_x1, s_y2, s_x2 = carry
        ncounts, ncms = [], []
        ns_s, ns_y1, ns_x1, ns_y2, ns_x2 = [], [], [], [], []
        for k in range(_G):
            count, cm = counts[k], cms[k]
            m = jnp.max(cm)
            active = (count < _K) & (m > _NEG)
            ch = jnp.min(jnp.where(cm == m, lane_i, big))
            base = ch * _CHUNK
            v8 = valid_refs[k][pl.ds(base, _CHUNK), :]
            j8 = jnp.min(jnp.where(v8 == m, flat8, big))
            r8 = j8 // _LANES
            col = j8 - r8 * _LANES
            hit = lane_i == col
            r = base + r8

            def comp(q):
                rowv = boxes_ref[q, pl.ds(r, 1), :]
                return jnp.sum(jnp.where(hit, rowv, 0.0))

            by1, bx1, by2, bx2 = comp(0), comp(1), comp(2), comp(3)

            yy1 = jnp.maximum(by1, s_y1[k])
            xx1 = jnp.maximum(bx1, s_x1[k])
            yy2 = jnp.minimum(by2, s_y2[k])
            xx2 = jnp.minimum(bx2, s_x2[k])
            inter = jnp.maximum(yy2 - yy1, 0.0) * jnp.maximum(xx2 - xx1, 0.0)
            area1 = (by2 - by1) * (bx2 - bx1)
            areas = (s_y2[k] - s_y1[k]) * (s_x2[k] - s_x1[k])
            iou = inter / jnp.maximum(area1 + areas - inter, 1e-9)
            keep = active & jnp.logical_not(jnp.max(iou) > _IOU_T)

            hit8 = (row8 == r8) & (col8 == col)
            v8n = jnp.where(hit8, _NEG, v8)
            valid_refs[k][pl.ds(base, _CHUNK), :] = v8n
            ncms.append(jnp.where(lane_i == ch, jnp.max(v8n), cm))

            slot = (lane_i == count) & keep
            ns_s.append(jnp.where(slot, m, s_s[k]))
            ns_y1.append(jnp.where(slot, by1, s_y1[k]))
            ns_x1.append(jnp.where(slot, bx1, s_x1[k]))
            ns_y2.append(jnp.where(slot, by2, s_y2[k]))
            ns_x2.append(jnp.where(slot, bx2, s_x2[k]))
            ncounts.append(count + keep.astype(jnp.int32))
        return (tuple(ncounts), tuple(ncms), tuple(ns_s), tuple(ns_y1),
                tuple(ns_x1), tuple(ns_y2), tuple(ns_x2))

    neg = jnp.full((1, _LANES), _NEG, jnp.float32)
    degv = jnp.full((1, _LANES), deg, jnp.float32)
    zero = jnp.int32(0)
    init = (tuple(zero for _ in range(_G)), tuple(cm0),
            tuple(neg for _ in range(_G)), tuple(degv for _ in range(_G)),
            tuple(degv for _ in range(_G)), tuple(degv for _ in range(_G)),
            tuple(degv for _ in range(_G)))
    counts, _, s_s, s_y1, s_x1, s_y2, s_x2 = jax.lax.while_loop(
        cond, pop, init)

    b0y1 = boxes_ref[0, 0:1, 0:1]
    b0x1 = boxes_ref[1, 0:1, 0:1]
    b0y2 = boxes_ref[2, 0:1, 0:1]
    b0x2 = boxes_ref[3, 0:1, 0:1]
    for k in range(_G):
        filled = lane_i < counts[k]
        sel_s_ref[k] = s_s[k]
        sel_b_ref[k] = jnp.concatenate([
            jnp.where(filled, s_y1[k], b0y1),
            jnp.where(filled, s_x1[k], b0x1),
            jnp.where(filled, s_y2[k], b0y2),
            jnp.where(filled, s_x2[k], b0x2),
        ], axis=0)


_MROWS = 8
_MCOLS = _C * _LANES // _MROWS


def _topk_body(scores_ref, planes_ref, boxes_ref, cls_ref, valid_ref):
    valid_ref[...] = scores_ref[...]
    row_i = jax.lax.broadcasted_iota(jnp.int32, (_MROWS, _MCOLS), 0)
    col_i = jax.lax.broadcasted_iota(jnp.int32, (_MROWS, _MCOLS), 1)
    flat_i = row_i * _MCOLS + col_i
    lane_i = jax.lax.broadcasted_iota(jnp.int32, (1, _MCOLS), 1)
    big = jnp.int32(2 ** 30)

    def step(i, _):
        v = valid_ref[...]
        m = jnp.max(v)
        j = jnp.min(jnp.where(v == m, flat_i, big))
        r = j // _MCOLS
        col = j - r * _MCOLS
        hit = lane_i == col

        vrow = valid_ref[pl.ds(r, 1), :]
        valid_ref[pl.ds(r, 1), :] = jnp.where(hit, _NEG, vrow)

        def comp(k):
            rowv = planes_ref[k, pl.ds(r, 1), :]
            return jnp.sum(jnp.where(hit, rowv, 0.0)).reshape(1, 1)

        boxes_ref[pl.ds(i, 1), :] = jnp.concatenate(
            [comp(0), comp(1), comp(2), comp(3)], axis=1)
        cls_id = (j // _LANES).reshape(1, 1)
        cls_ref[pl.ds(i, 1), :] = jnp.broadcast_to(cls_id, (1, 4))
        return 0

    jax.lax.fori_loop(0, _OUT, step, 0)


def kernel(roi_bboxes_txtytwth, roi_score, rpn_proposals_bboxes):
    npad = _ROWS * _LANES - _N

    props = jnp.pad(rpn_proposals_bboxes, ((0, npad), (0, 0)))
    props = props.T.reshape(4, _ROWS, _LANES)
    deltas = jnp.pad(roi_bboxes_txtytwth, ((0, npad), (0, 0)))
    deltas = deltas.T.reshape(4, _ROWS, _LANES)
    scores = jnp.pad(roi_score.T, ((0, 0), (0, npad)),
                     constant_values=_NEG).reshape(_C, _ROWS, _LANES)

    sel_s, sel_b = pl.pallas_call(
        _nms_body,
        grid=(_C // _G,),
        in_specs=[
            pl.BlockSpec((4, _ROWS, _LANES), lambda c: (0, 0, 0)),
            pl.BlockSpec((4, _ROWS, _LANES), lambda c: (0, 0, 0)),
            pl.BlockSpec((_G, _ROWS, _LANES), lambda c: (c, 0, 0)),
        ],
        out_specs=[
            pl.BlockSpec((_G, 1, _LANES), lambda c: (c, 0, 0)),
            pl.BlockSpec((_G, 4, _LANES), lambda c: (c, 0, 0)),
        ],
        out_shape=[
            jax.ShapeDtypeStruct((_C, 1, _LANES), jnp.float32),
            jax.ShapeDtypeStruct((_C, 4, _LANES), jnp.float32),
        ],
        scratch_shapes=[pltpu.VMEM((4, _ROWS, _LANES), jnp.float32)] + [
            pltpu.VMEM((_ROWS, _LANES), jnp.float32) for _ in range(_G)],
    )(props, deltas, scores)

    merged_s = sel_s.reshape(_MROWS, _MCOLS)
    merged_b = sel_b.transpose(1, 0, 2).reshape(4, _MROWS, _MCOLS)

    boxes_out, cls_out = pl.pallas_call(
        _topk_body,
        out_shape=[
            jax.ShapeDtypeStruct((_OUT + 4, 4), jnp.float32),
            jax.ShapeDtypeStruct((_OUT + 4, 4), jnp.int32),
        ],
        scratch_shapes=[pltpu.VMEM((_MROWS, _MCOLS), jnp.float32)],
    )(merged_s, merged_b)

    return boxes_out[:_OUT], cls_out[:_OUT, 0]
